# baseline (speedup 1.0000x reference)
import jax
import jax.numpy as jnp
from jax import lax
from jax.experimental import pallas as pl
from jax.experimental.pallas import tpu as pltpu
from jax.experimental.pallas import tpu_sc as plsc

_B = 4096
_D = 128
_NNEG = 64
_REG = 0.01

_info = plsc.get_sparse_core_info()
_NC = _info.num_cores
_NS = _info.num_subcores
_L = _info.num_lanes
_NW = _NC * _NS
_BPW = _B // _NW
_NV = _D // _L
_G = 1
_NP = _BPW // _G


def _log1p_series(u):
    z = u / (2.0 + u)
    z2 = z * z
    p = 1.0 / 13.0
    for c in (11.0, 9.0, 7.0, 5.0, 3.0):
        p = p * z2 + 1.0 / c
    p = p * z2 + 1.0
    return 2.0 * z * p


def _log_sig(x):
    return jnp.minimum(x, 0.0) - _log1p_series(jnp.exp(-jnp.abs(x)))


def _sc_body(flat_hbm, rels_hbm, nidx_hbm, relw_hbm,
             out_hbm,
             acc_sh,
             relrow_v, rels_v, nidx_v, ent_c,
             buf0, buf1, acc_v, idx16, outrow_v,
             sem0, sem1, sem_rel, sem_ent):
    sid = lax.axis_index("s")
    cid = lax.axis_index("c")
    wid = sid * _NC + cid
    base = wid * _BPW

    pltpu.sync_copy(rels_hbm.at[pl.ds(base, _BPW)], rels_v)
    rel_cp = pltpu.async_copy(relw_hbm.at[rels_v], relrow_v, sem_rel)
    pltpu.sync_copy(nidx_hbm.at[pl.ds(wid * _NP, _NP)], nidx_v)

    zrow = jnp.zeros((_L,), jnp.float32)
    for r in range(_L):
        acc_v[r, :] = zrow
    idx16[...] = lax.iota(jnp.int32, _L)

    @pl.when(sid == 0)
    def _zero_shared():
        pltpu.sync_copy(acc_v, acc_sh)

    plsc.subcore_barrier()

    bufs = (buf0, buf1)
    sems = (sem0, sem1)

    def cp(pp, par):
        return pltpu.make_async_copy(
            flat_hbm.at[nidx_v.at[pp]], bufs[par], sems[par])

    cp(0, 0).start()
    cp(1, 1).start()
    pltpu.sync_copy(flat_hbm.at[pl.ds(2 * base, 4 * _L)], ent_c)
    rel_cp.wait()

    zero = jnp.zeros((_L,), jnp.float32)
    lane = lax.iota(jnp.int32, _L)
    rots = [((lane + k) & (_L - 1)).reshape(_L, 1) for k in (8, 4, 2, 1)]
    _dnums = lax.GatherDimensionNumbers(
        offset_dims=(), collapsed_slice_dims=(0,), start_index_map=(0,))

    def lane_sum(x):
        for perm in rots:
            x = x + lax.gather(x, perm, _dnums, (1,),
                               mode=lax.GatherScatterMode.PROMISE_IN_BOUNDS)
        return x

    @pl.loop(0, _NP, step=2, init_carry=zero)
    def pos_pend(p, pos_pend):
        for par in range(2):
            pp = p + par
            cp(pp, par).wait()
            buf = bufs[par]

            @pl.when(jnp.logical_and((pp & 31) == 0, pp > 0))
            def _reload_ent():
                pltpu.sync_copy(
                    flat_hbm.at[pl.ds(pl.multiple_of(2 * base + 2 * pp, 8),
                                      4 * _L)], ent_c)

            for g in range(_G):
                bb = _G * pp + g
                lrow = 2 * (pp & 31) + 2 * g

                relv = [relrow_v[bb, pl.ds(v * _L, _L)] for v in range(_NV)]

                r2 = zero
                for v in range(_NV):
                    r2 = r2 + relv[v] * relv[v]
                acc_v[2, :] = acc_v[2, :] + r2

                pacc = zero
                e2 = zero
                for v in range(_NV):
                    h = ent_c[lrow, pl.ds(v * _L, _L)]
                    t = ent_c[lrow + 1, pl.ds(v * _L, _L)]
                    pacc = pacc + h * relv[v] * t
                    e2 = e2 + h * h + t * t
                acc_v[3, :] = acc_v[3, :] + e2
                pos_pend = jnp.where(lane == (bb & (_L - 1)),
                                     lane_sum(pacc), pos_pend)

                if g == _G - 1:
                    @pl.when((bb & (_L - 1)) == (_L - 1))
                    def _flush_pos():
                        acc_v[1, :] = acc_v[1, :] + _log_sig(pos_pend)

                for c in range(_NNEG // _L):
                    @plsc.parallel_loop(0, _L, carry=zero, unroll=4)
                    def pending(n, pending):
                        nn = 2 * (g * _NNEG + c * _L + n)
                        q = [buf[nn, pl.ds(v * _L, _L)]
                             * buf[nn + 1, pl.ds(v * _L, _L)]
                             * relv[v] for v in range(_NV)]
                        s = (((q[0] + q[1]) + (q[2] + q[3]))
                             + ((q[4] + q[5]) + (q[6] + q[7])))
                        return jnp.where(lane == n, lane_sum(s), pending)

                    acc_v[0, :] = acc_v[0, :] + _log_sig(-pending)

            @pl.when(pp + 2 < _NP)
            def _prefetch():
                cp(pp + 2, par).start()
        return pos_pend

    plsc.subcore_barrier()
    pltpu.sync_copy(acc_v, acc_sh.at[idx16], add=True)
    plsc.subcore_barrier()

    @pl.when(sid == 0)
    def _finalize():
        pltpu.sync_copy(acc_sh, acc_v)
        rows = [lane_sum(acc_v[r, :]) for r in range(4)]
        o = jnp.where(lane == 0, rows[0], zero)
        o = jnp.where(lane == 1, rows[1], o)
        o = jnp.where(lane == 2, rows[2], o)
        o = jnp.where(lane == 3, rows[3], o)
        outrow_v[...] = o
        pltpu.sync_copy(outrow_v, out_hbm.at[cid])


def _sc_loss(flat, rels, nidx, relw):
    mesh = plsc.VectorSubcoreMesh(core_axis_name="c", subcore_axis_name="s")
    return pl.kernel(
        _sc_body,
        out_type=jax.ShapeDtypeStruct((_NC, _L), jnp.float32),
        mesh=mesh,
        scratch_types=[
            pltpu.VMEM_SHARED((_L, _L), jnp.float32),
            pltpu.VMEM((_BPW, _D), jnp.float32),
            pltpu.VMEM((_BPW,), jnp.int32),
            pltpu.VMEM((_NP, _G * 2 * _NNEG), jnp.int32),
            pltpu.VMEM((4 * _L, _D), jnp.float32),
            pltpu.VMEM((_G * 2 * _NNEG, _D), jnp.float32),
            pltpu.VMEM((_G * 2 * _NNEG, _D), jnp.float32),
            pltpu.VMEM((_L, _L), jnp.float32),
            pltpu.VMEM((_L,), jnp.int32),
            pltpu.VMEM((_L,), jnp.float32),
            pltpu.SemaphoreType.DMA,
            pltpu.SemaphoreType.DMA,
            pltpu.SemaphoreType.DMA,
            pltpu.SemaphoreType.DMA,
        ],
    )(flat, rels, nidx, relw)


def kernel(ent_embs, rels, neg_idx, rel_emb_weight):
    ent = ent_embs.astype(jnp.float32)
    flat = ent.reshape(2 * _B, _D)
    rels1 = rels.reshape(_B).astype(jnp.int32)
    nidx = neg_idx.astype(jnp.int32).reshape(_NW * _NP, _G * 2 * _NNEG)
    relw = rel_emb_weight.astype(jnp.float32)
    parts = _sc_loss(flat, rels1, nidx, relw)
    neg_sum = parts[0, 0] + parts[1, 0]
    pos_sum = parts[0, 1] + parts[1, 1]
    rel_sq = parts[0, 2] + parts[1, 2]
    ent_sq = parts[0, 3] + parts[1, 3]
    neg_loss = -neg_sum / (_B * _NNEG)
    pos_loss = -pos_sum / _B
    model_loss = (pos_loss + neg_loss) * 0.5
    reg = _REG * ((ent_sq + rel_sq) / (_B * _D)) / 3.0
    return model_loss + reg


# speedup vs baseline: 1.0485x; 1.0485x over previous
import jax
import jax.numpy as jnp
import numpy as np
from jax import lax
from jax.experimental import pallas as pl
from jax.experimental.pallas import tpu as pltpu
from jax.experimental.pallas import tpu_sc as plsc

_B = 4096
_D = 128
_NNEG = 64
_REG = 0.01

_info = plsc.get_sparse_core_info()
_NC = _info.num_cores
_NS = _info.num_subcores
_L = _info.num_lanes
_NW = _NC * _NS
_BPW = _B // _NW
_NV = _D // _L
_G = 1
_NP = _BPW // _G


def _sc_body(flat_hbm, rels_hbm, nidx_hbm, relw_hbm,
             neg_out, relrow_out,
             relrow_v, rels_v, nidx_v,
             buf0, buf1, scores_v,
             sem0, sem1, sem_rel):
    wid = lax.axis_index("s") * _NC + lax.axis_index("c")
    base = wid * _BPW

    pltpu.sync_copy(rels_hbm.at[pl.ds(base, _BPW)], rels_v)
    rel_cp = pltpu.async_copy(relw_hbm.at[rels_v], relrow_v, sem_rel)
    pltpu.sync_copy(nidx_hbm.at[pl.ds(wid * _NP, _NP)], nidx_v)

    bufs = (buf0, buf1)
    sems = (sem0, sem1)

    def cp(pp, par):
        return pltpu.make_async_copy(
            flat_hbm.at[nidx_v.at[pp]], bufs[par], sems[par])

    cp(0, 0).start()
    cp(1, 1).start()
    rel_cp.wait()

    zero = jnp.zeros((_L,), jnp.float32)
    lane = lax.iota(jnp.int32, _L)
    rots = [((lane + k) & (_L - 1)).reshape(_L, 1) for k in (8, 4, 2, 1)]
    _dnums = lax.GatherDimensionNumbers(
        offset_dims=(), collapsed_slice_dims=(0,), start_index_map=(0,))

    def lane_sum(x):
        for perm in rots:
            x = x + lax.gather(x, perm, _dnums, (1,),
                               mode=lax.GatherScatterMode.PROMISE_IN_BOUNDS)
        return x

    @pl.loop(0, _NP, step=2)
    def _p_loop(p):
        for par in range(2):
            pp = p + par
            cp(pp, par).wait()
            buf = bufs[par]

            for g in range(_G):
                bb = _G * pp + g

                relv = [relrow_v[bb, pl.ds(v * _L, _L)] for v in range(_NV)]

                for c in range(_NNEG // _L):
                    @plsc.parallel_loop(0, _L, carry=zero, unroll=4)
                    def pending(n, pending):
                        nn = 2 * (g * _NNEG + c * _L + n)
                        q = [buf[nn, pl.ds(v * _L, _L)]
                             * buf[nn + 1, pl.ds(v * _L, _L)]
                             * relv[v] for v in range(_NV)]
                        s = (((q[0] + q[1]) + (q[2] + q[3]))
                             + ((q[4] + q[5]) + (q[6] + q[7])))
                        return jnp.where(lane == n, lane_sum(s), pending)

                    scores_v[bb, pl.ds(c * _L, _L)] = pending

            @pl.when(pp + 2 < _NP)
            def _prefetch():
                cp(pp + 2, par).start()

    pltpu.sync_copy(scores_v, neg_out.at[pl.ds(base, _BPW)])
    pltpu.sync_copy(relrow_v, relrow_out.at[pl.ds(base, _BPW)])


def _sc_scores(flat, rels, nidx, relw):
    mesh = plsc.VectorSubcoreMesh(core_axis_name="c", subcore_axis_name="s")
    return pl.kernel(
        _sc_body,
        out_type=(
            jax.ShapeDtypeStruct((_B, _NNEG), jnp.float32),
            jax.ShapeDtypeStruct((_B, _D), jnp.float32),
        ),
        mesh=mesh,
        scratch_types=[
            pltpu.VMEM((_BPW, _D), jnp.float32),
            pltpu.VMEM((_BPW,), jnp.int32),
            pltpu.VMEM((_NP, _G * 2 * _NNEG), jnp.int32),
            pltpu.VMEM((_G * 2 * _NNEG, _D), jnp.float32),
            pltpu.VMEM((_G * 2 * _NNEG, _D), jnp.float32),
            pltpu.VMEM((_BPW, _NNEG), jnp.float32),
            pltpu.SemaphoreType.DMA,
            pltpu.SemaphoreType.DMA,
            pltpu.SemaphoreType.DMA,
        ],
    )(flat, rels, nidx, relw)


def _log_sigmoid(x):
    return jnp.minimum(x, 0.0) - jnp.log1p(jnp.exp(-jnp.abs(x)))


def _tc_body(neg_ref, relrow_ref, ent_ref, out_ref):
    neg = neg_ref[...]
    rel = relrow_ref[...]
    ent = ent_ref[...]
    heads = ent[:, 0, :]
    tails = ent[:, 1, :]
    pos = jnp.sum(heads * rel * tails, axis=-1)
    neg_loss = -jnp.sum(_log_sigmoid(-neg)) / (_B * _NNEG)
    pos_loss = -jnp.sum(_log_sigmoid(pos)) / _B
    model_loss = (pos_loss + neg_loss) * 0.5
    ent_sq = jnp.sum(ent * ent) / (_B * _D)
    rel_sq = jnp.sum(rel * rel) / (_B * _D)
    reg = _REG * ((ent_sq + rel_sq) / 3.0)
    out_ref[...] = jnp.full((1, 1), 0.0, jnp.float32) + model_loss + reg


def _tc_finish(neg_scores, relrows, ent_embs):
    out = pl.pallas_call(
        _tc_body,
        out_shape=jax.ShapeDtypeStruct((1, 1), jnp.float32),
    )(neg_scores, relrows, ent_embs)
    return out[0, 0]


def kernel(ent_embs, rels, neg_idx, rel_emb_weight):
    ent = ent_embs.astype(jnp.float32)
    flat = ent.reshape(2 * _B, _D)
    rels1 = rels.reshape(_B).astype(jnp.int32)
    nidx = neg_idx.astype(jnp.int32).reshape(_NW * _NP, _G * 2 * _NNEG)
    relw = rel_emb_weight.astype(jnp.float32)
    neg_scores, relrows = _sc_scores(flat, rels1, nidx, relw)
    return _tc_finish(neg_scores, relrows, ent)
